# baseline (speedup 1.0000x reference)
import functools

import jax
import jax.numpy as jnp
from jax import lax
from jax.experimental import pallas as pl
from jax.experimental.pallas import tpu as pltpu
from jax.experimental.pallas import tpu_sc as plsc

N_PROT = 50000
N_DRUG = 5000
D_PROT = 128
D1 = 64
D2 = 32
D_OUT = 32

BM = 512
NP = 50176
NDP = 5120

E_PP = 800000
E_PD = 500000
CK = 128
IB = 4
PPCH = 6400
PDCH = 4096
EP = PPCH * CK
EPD = PDCH * CK

_MESH = plsc.VectorSubcoreMesh(core_axis_name="c", subcore_axis_name="s")


def _sc_counts(ppd, pdd, znp):
    chp = PPCH // 32
    chd = PDCH // 32
    rp = NP // 16
    rd = NDP // 16

    @functools.partial(
        pl.kernel, mesh=_MESH,
        out_type=[jax.ShapeDtypeStruct((2 * NP,), jnp.float32),
                  jax.ShapeDtypeStruct((2 * NDP,), jnp.float32)],
        scratch_types=[pltpu.VMEM((chp, CK), jnp.int32),
                       pltpu.VMEM((chd, CK), jnp.int32),
                       pltpu.VMEM((CK,), jnp.float32),
                       pltpu.VMEM((rp,), jnp.float32),
                       pltpu.VMEM_SHARED((NP,), jnp.float32),
                       pltpu.VMEM_SHARED((NDP,), jnp.float32),
                       pltpu.SemaphoreType.DMA],
        compiler_params=pltpu.CompilerParams(use_tc_tiling_on_sc=False),
    )
    def k(ppd_h, pdd_h, znp_h, outp_h, outd_h,
          ppv, pdv, ones, stg, accp, accd, ssem):
        c = lax.axis_index("c")
        s = lax.axis_index("s")
        w = c * 16 + s
        pltpu.sync_copy(znp_h, stg)
        pltpu.sync_copy(stg, accp.at[pl.ds(s * rp, rp)])
        pltpu.sync_copy(stg.at[pl.ds(0, rd)], accd.at[pl.ds(s * rd, rd)])
        for i in range(CK // 16):
            ones[pl.ds(i * 16, 16)] = jnp.ones((16,), jnp.float32)
        pltpu.sync_copy(ppd_h.at[pl.ds(w * chp, chp)], ppv)
        pltpu.sync_copy(pdd_h.at[pl.ds(w * chd, chd)], pdv)
        plsc.subcore_barrier()

        def body_p(j, carry):
            pltpu.async_copy(ones, accp.at[ppv.at[j]], ssem, add=True)

            @pl.when(j >= 8)
            def _():
                pltpu.make_async_copy(ones, accp.at[ppv.at[0]], ssem).wait()
            return carry
        lax.fori_loop(0, chp, body_p, 0)
        for _ in range(8):
            pltpu.make_async_copy(ones, accp.at[ppv.at[0]], ssem).wait()

        def body_d(j, carry):
            pltpu.async_copy(ones, accd.at[pdv.at[j]], ssem, add=True)

            @pl.when(j >= 8)
            def _():
                pltpu.make_async_copy(ones, accd.at[pdv.at[0]], ssem).wait()
            return carry
        lax.fori_loop(0, chd, body_d, 0)
        for _ in range(8):
            pltpu.make_async_copy(ones, accd.at[pdv.at[0]], ssem).wait()
        plsc.subcore_barrier()
        pltpu.sync_copy(accp.at[pl.ds(s * rp, rp)], stg)
        pltpu.sync_copy(stg, outp_h.at[pl.ds(c * NP + s * rp, rp)])
        pltpu.sync_copy(accd.at[pl.ds(s * rd, rd)], stg.at[pl.ds(0, rd)])
        pltpu.sync_copy(stg.at[pl.ds(0, rd)],
                        outd_h.at[pl.ds(c * NDP + s * rd, rd)])

    return k(ppd, pdd, znp)


def _make_gs(ch, src_stride, dst_stride, acc_rows):
    rp = acc_rows // 16
    nz = 8
    zr = rp // nz
    nbt = ch // IB

    def build(tbl, srci, dsti, zeros):
        @functools.partial(
            pl.kernel, mesh=_MESH,
            out_type=jax.ShapeDtypeStruct((2 * acc_rows, D2), jnp.float32),
            scratch_types=[pltpu.VMEM((2, IB, CK), jnp.int32),
                           pltpu.VMEM((2, IB, CK), jnp.int32),
                           pltpu.VMEM((2, CK, D2), jnp.float32),
                           pltpu.VMEM((zr, D2), jnp.float32),
                           pltpu.VMEM_SHARED((acc_rows, D2), jnp.float32),
                           pltpu.SemaphoreType.DMA,
                           pltpu.SemaphoreType.DMA,
                           pltpu.SemaphoreType.DMA,
                           pltpu.SemaphoreType.DMA,
                           pltpu.SemaphoreType.DMA],
            compiler_params=pltpu.CompilerParams(use_tc_tiling_on_sc=False),
        )
        def k(tbl_h, src_h, dst_h, z_h, out_h, srcb, dstb, rows, stg, acc,
              gs0, gs1, ss0, ss1, isem):
            c = lax.axis_index("c")
            s = lax.axis_index("s")
            pltpu.sync_copy(z_h, stg)
            for i in range(nz):
                pltpu.sync_copy(stg, acc.at[pl.ds(s * rp + i * zr, zr)])
            so = c * src_stride + s * ch
            do = c * dst_stride + s * ch
            pltpu.sync_copy(src_h.at[pl.ds(so, IB)], srcb.at[0])
            pltpu.sync_copy(dst_h.at[pl.ds(do, IB)], dstb.at[0])
            plsc.subcore_barrier()
            pltpu.async_copy(tbl_h.at[srcb.at[0].at[0]], rows.at[0], gs0)

            def batch2(bb2, carry):
                for bp in range(2):
                    bb = bb2 * 2 + bp
                    for kk in range(IB):
                        b = kk % 2
                        gsb = gs0 if b == 0 else gs1
                        gsn = gs1 if b == 0 else gs0
                        ssb = ss0 if b == 0 else ss1
                        ssn = ss1 if b == 0 else ss0
                        if kk == 0:
                            @pl.when(bb > 0)
                            def _():
                                pltpu.make_async_copy(
                                    rows.at[1 - b],
                                    acc.at[dstb.at[bp].at[kk]], ssn).wait()

                            @pl.when(bb + 1 < nbt)
                            def _():
                                pltpu.async_copy(
                                    src_h.at[pl.ds(so + (bb + 1) * IB, IB)],
                                    srcb.at[1 - bp], isem)
                                pltpu.async_copy(
                                    dst_h.at[pl.ds(do + (bb + 1) * IB, IB)],
                                    dstb.at[1 - bp], isem)
                        else:
                            pltpu.make_async_copy(
                                rows.at[1 - b],
                                acc.at[dstb.at[bp].at[kk]], ssn).wait()
                        if kk == IB - 1:
                            @pl.when(bb + 1 < nbt)
                            def _():
                                pltpu.make_async_copy(
                                    src_h.at[pl.ds(so + (bb + 1) * IB, IB)],
                                    srcb.at[1 - bp], isem).wait()
                                pltpu.make_async_copy(
                                    dst_h.at[pl.ds(do + (bb + 1) * IB, IB)],
                                    dstb.at[1 - bp], isem).wait()
                                pltpu.async_copy(
                                    tbl_h.at[srcb.at[1 - bp].at[0]],
                                    rows.at[1 - b], gsn)
                        else:
                            pltpu.async_copy(tbl_h.at[srcb.at[bp].at[kk + 1]],
                                             rows.at[1 - b], gsn)
                        pltpu.make_async_copy(tbl_h.at[srcb.at[bp].at[kk]],
                                              rows.at[b], gsb).wait()
                        pltpu.async_copy(rows.at[b],
                                         acc.at[dstb.at[bp].at[kk]], ssb,
                                         add=True)
                return carry

            lax.fori_loop(0, nbt // 2, batch2, 0)
            pltpu.make_async_copy(rows.at[1], acc.at[dstb.at[1].at[0]],
                                  ss1).wait()
            plsc.subcore_barrier()
            for i in range(nz):
                pltpu.sync_copy(acc.at[pl.ds(s * rp + i * zr, zr)], stg)
                pltpu.sync_copy(
                    stg, out_h.at[pl.ds(c * acc_rows + s * rp + i * zr, zr)])

        return k(tbl, srci, dsti, zeros)

    return build


_gs_l1 = _make_gs(PPCH // 16, PPCH, 0, NP)
_gs_l2 = _make_gs(PPCH // 32, PPCH // 2, PPCH // 2, NP)
_gs_pd = _make_gs(PDCH // 32, PDCH // 2, PDCH // 2, NDP)


def _tc1_body(x_ref, c0_ref, c1_ref, w_ref, g_ref, dinv_ref):
    drow = lax.rsqrt(1.0 + c0_ref[0, 0] + c1_ref[0, 0])
    dinv_ref[0, 0] = drow
    xs = x_ref[...] * drow.reshape(BM, 1)
    g_ref[0] = jnp.dot(xs, w_ref[0], preferred_element_type=jnp.float32)
    g_ref[1] = jnp.dot(xs, w_ref[1], preferred_element_type=jnp.float32)


def _tc1(x_pad, cntr, W1):
    nb = NP // BM
    f = pl.pallas_call(
        _tc1_body,
        grid=(nb,),
        in_specs=[
            pl.BlockSpec((BM, D_PROT), lambda j: (j, 0)),
            pl.BlockSpec((1, 1, BM), lambda j: (j, 0, 0)),
            pl.BlockSpec((1, 1, BM), lambda j: (nb + j, 0, 0)),
            pl.BlockSpec((2, D_PROT, D2), lambda j: (0, 0, 0)),
        ],
        out_specs=[
            pl.BlockSpec((2, BM, D2), lambda j: (0, j, 0)),
            pl.BlockSpec((1, 1, BM), lambda j: (j, 0, 0)),
        ],
        out_shape=[
            jax.ShapeDtypeStruct((2, NP, D2), jnp.float32),
            jax.ShapeDtypeStruct((nb, 1, BM), jnp.float32),
        ],
    )
    W1s = jnp.stack([W1[:, :D2], W1[:, D2:]])
    g3, dinvr = f(x_pad, cntr, cntr, W1s)
    return g3.reshape(2 * NP, D2), dinvr


def _tc2_body(s1a_ref, s1b_ref, g1a_ref, g1b_ref, dinv_ref, b1_ref, w2_ref,
              g2_ref):
    dinv = dinv_ref[0, 0].reshape(BM, 1)
    ha = jax.nn.relu(dinv * (s1a_ref[...] + g1a_ref[...]) + b1_ref[0, 0])
    hb = jax.nn.relu(dinv * (s1b_ref[...] + g1b_ref[...]) + b1_ref[0, 1])
    g2_ref[...] = (jnp.dot(ha * dinv, w2_ref[0],
                           preferred_element_type=jnp.float32)
                   + jnp.dot(hb * dinv, w2_ref[1],
                             preferred_element_type=jnp.float32))


def _tc2(s1cat, g1cat, dinvr, b1, W2):
    nb = NP // BM
    f = pl.pallas_call(
        _tc2_body,
        grid=(nb,),
        in_specs=[
            pl.BlockSpec((BM, D2), lambda j: (j, 0)),
            pl.BlockSpec((BM, D2), lambda j: (nb + j, 0)),
            pl.BlockSpec((BM, D2), lambda j: (j, 0)),
            pl.BlockSpec((BM, D2), lambda j: (nb + j, 0)),
            pl.BlockSpec((1, 1, BM), lambda j: (j, 0, 0)),
            pl.BlockSpec((1, 2, D2), lambda j: (0, 0, 0)),
            pl.BlockSpec((2, D2, D2), lambda j: (0, 0, 0)),
        ],
        out_specs=pl.BlockSpec((BM, D2), lambda j: (j, 0)),
        out_shape=jax.ShapeDtypeStruct((NP, D2), jnp.float32),
    )
    return f(s1cat, s1cat, g1cat, g1cat, dinvr,
             b1.reshape(1, 2, D2), W2.reshape(2, D2, D2))


def _tc3_body(s2a_ref, s2b_ref, g2_ref, dinv_ref, b2_ref, hf_ref):
    dinv = dinv_ref[0, 0].reshape(BM, 1)
    hf_ref[...] = dinv * (s2a_ref[...] + s2b_ref[...] + g2_ref[...]) \
        + b2_ref[...]


def _tc3(s2cat, g2, dinvr, b2):
    nb = NP // BM
    f = pl.pallas_call(
        _tc3_body,
        grid=(nb,),
        in_specs=[
            pl.BlockSpec((BM, D2), lambda j: (j, 0)),
            pl.BlockSpec((BM, D2), lambda j: (nb + j, 0)),
            pl.BlockSpec((BM, D2), lambda j: (j, 0)),
            pl.BlockSpec((1, 1, BM), lambda j: (j, 0, 0)),
            pl.BlockSpec((1, D2), lambda j: (0, 0)),
        ],
        out_specs=pl.BlockSpec((BM, D2), lambda j: (j, 0)),
        out_shape=jax.ShapeDtypeStruct((N_PROT, D2), jnp.float32),
    )
    return f(s2cat, s2cat, g2, dinvr, b2.reshape(1, D2))


def _tc4_body(sa_ref, sb_ref, ca_ref, cb_ref, xd_ref, wl_ref, bl_ref, wr_ref,
              out_ref):
    crow = ca_ref[0, 0] + cb_ref[0, 0]
    cnt = jnp.maximum(crow, 1.0).reshape(BM, 1)
    mean = (sa_ref[...] + sb_ref[...]) / cnt
    out_ref[...] = (jnp.dot(mean, wl_ref[...],
                            preferred_element_type=jnp.float32)
                    + bl_ref[...]
                    + jnp.dot(xd_ref[...], wr_ref[...],
                              preferred_element_type=jnp.float32))


def _tc4(sumcat, cntdr, xd_pad, Wl, bl, Wr):
    nb = NDP // BM
    f = pl.pallas_call(
        _tc4_body,
        grid=(nb,),
        in_specs=[
            pl.BlockSpec((BM, D2), lambda j: (j, 0)),
            pl.BlockSpec((BM, D2), lambda j: (nb + j, 0)),
            pl.BlockSpec((1, 1, BM), lambda j: (j, 0, 0)),
            pl.BlockSpec((1, 1, BM), lambda j: (nb + j, 0, 0)),
            pl.BlockSpec((BM, D_PROT), lambda j: (j, 0)),
            pl.BlockSpec((D2, D_OUT), lambda j: (0, 0)),
            pl.BlockSpec((1, D_OUT), lambda j: (0, 0)),
            pl.BlockSpec((D_PROT, D_OUT), lambda j: (0, 0)),
        ],
        out_specs=pl.BlockSpec((BM, D_OUT), lambda j: (j, 0)),
        out_shape=jax.ShapeDtypeStruct((N_DRUG, D_OUT), jnp.float32),
    )
    return f(sumcat, sumcat, cntdr, cntdr, xd_pad, Wl, bl.reshape(1, D_OUT),
             Wr)


def kernel(x_prot, x_drug, pp_edge_index, pd_src, pd_dst,
           W1, b1, W2, b2, Wl, bl, Wr):
    src = pp_edge_index[0]
    dst = pp_edge_index[1]

    x_pad = jnp.zeros((NP, D_PROT), jnp.float32).at[:N_PROT].set(x_prot)
    xd_pad = jnp.zeros((NDP, D_PROT), jnp.float32).at[:N_DRUG].set(x_drug)

    padp = EP - E_PP
    trash_p = (N_PROT + (jnp.arange(padp, dtype=jnp.int32)
                         % (NP - N_PROT))).astype(jnp.int32)
    srcp = jnp.concatenate([src, trash_p]).reshape(PPCH, CK)
    dstp = jnp.concatenate([dst, trash_p]).reshape(PPCH, CK)
    src2 = jnp.concatenate([srcp, srcp + NP], axis=0)

    padd = EPD - E_PD
    trash_ds = (jnp.arange(padd, dtype=jnp.int32) % 176).astype(jnp.int32)
    trash_dd = (N_DRUG + (jnp.arange(padd, dtype=jnp.int32)
                          % (NDP - N_DRUG))).astype(jnp.int32)
    pdsrcp = jnp.concatenate([pd_src, trash_ds]).reshape(PDCH, CK)
    pddstp = jnp.concatenate([pd_dst, trash_dd]).reshape(PDCH, CK)

    znp1 = jnp.zeros((NP // 16,), jnp.float32)
    z32np = jnp.zeros((NP // 128, D2), jnp.float32)
    z32nd = jnp.zeros((NDP // 128, D2), jnp.float32)

    cntp, cntd = _sc_counts(dstp, pddstp, znp1)
    cntr = cntp.reshape(2 * NP // BM, 1, BM)

    g1cat, dinvr = _tc1(x_pad, cntr, W1)

    s1cat = _gs_l1(g1cat, src2, dstp, z32np)

    g2 = _tc2(s1cat, g1cat, dinvr, b1, W2)

    s2cat = _gs_l2(g2, srcp, dstp, z32np)

    hf = _tc3(s2cat, g2, dinvr, b2)

    sumcat = _gs_pd(hf, pdsrcp, pddstp, z32nd)

    cntdr = cntd.reshape(2 * NDP // BM, 1, BM)
    dti = _tc4(sumcat, cntdr, xd_pad, Wl, bl, Wr)

    return (hf, dti)


# speedup vs baseline: 1.0510x; 1.0510x over previous
import functools

import jax
import jax.numpy as jnp
from jax import lax
from jax.experimental import pallas as pl
from jax.experimental.pallas import tpu as pltpu
from jax.experimental.pallas import tpu_sc as plsc

N_PROT = 50000
N_DRUG = 5000
D_PROT = 128
D1 = 64
D2 = 32
D_OUT = 32

BM = 512
NP = 50176
NDP = 5120

E_PP = 800000
E_PD = 500000
CK = 128
IB = 4
PPCH = 6400
PDCH = 4096
EP = PPCH * CK
EPD = PDCH * CK

_MESH = plsc.VectorSubcoreMesh(core_axis_name="c", subcore_axis_name="s")


def _sc_counts(ppd, pdd, znp):
    chp = PPCH // 32
    chd = PDCH // 32
    rp = NP // 16
    rd = NDP // 16

    @functools.partial(
        pl.kernel, mesh=_MESH,
        out_type=[jax.ShapeDtypeStruct((2 * NP,), jnp.float32),
                  jax.ShapeDtypeStruct((2 * NDP,), jnp.float32)],
        scratch_types=[pltpu.VMEM((chp, CK), jnp.int32),
                       pltpu.VMEM((chd, CK), jnp.int32),
                       pltpu.VMEM((CK,), jnp.float32),
                       pltpu.VMEM((rp,), jnp.float32),
                       pltpu.VMEM_SHARED((NP,), jnp.float32),
                       pltpu.VMEM_SHARED((NDP,), jnp.float32),
                       pltpu.SemaphoreType.DMA],
        compiler_params=pltpu.CompilerParams(use_tc_tiling_on_sc=False),
    )
    def k(ppd_h, pdd_h, znp_h, outp_h, outd_h,
          ppv, pdv, ones, stg, accp, accd, ssem):
        c = lax.axis_index("c")
        s = lax.axis_index("s")
        w = c * 16 + s
        pltpu.sync_copy(znp_h, stg)
        pltpu.sync_copy(stg, accp.at[pl.ds(s * rp, rp)])
        pltpu.sync_copy(stg.at[pl.ds(0, rd)], accd.at[pl.ds(s * rd, rd)])
        for i in range(CK // 16):
            ones[pl.ds(i * 16, 16)] = jnp.ones((16,), jnp.float32)
        pltpu.sync_copy(ppd_h.at[pl.ds(w * chp, chp)], ppv)
        pltpu.sync_copy(pdd_h.at[pl.ds(w * chd, chd)], pdv)
        plsc.subcore_barrier()

        def body_p(j, carry):
            pltpu.async_copy(ones, accp.at[ppv.at[j]], ssem, add=True)

            @pl.when(j >= 8)
            def _():
                pltpu.make_async_copy(ones, accp.at[ppv.at[0]], ssem).wait()
            return carry
        lax.fori_loop(0, chp, body_p, 0)
        for _ in range(8):
            pltpu.make_async_copy(ones, accp.at[ppv.at[0]], ssem).wait()

        def body_d(j, carry):
            pltpu.async_copy(ones, accd.at[pdv.at[j]], ssem, add=True)

            @pl.when(j >= 8)
            def _():
                pltpu.make_async_copy(ones, accd.at[pdv.at[0]], ssem).wait()
            return carry
        lax.fori_loop(0, chd, body_d, 0)
        for _ in range(8):
            pltpu.make_async_copy(ones, accd.at[pdv.at[0]], ssem).wait()
        plsc.subcore_barrier()
        pltpu.sync_copy(accp.at[pl.ds(s * rp, rp)], stg)
        pltpu.sync_copy(stg, outp_h.at[pl.ds(c * NP + s * rp, rp)])
        pltpu.sync_copy(accd.at[pl.ds(s * rd, rd)], stg.at[pl.ds(0, rd)])
        pltpu.sync_copy(stg.at[pl.ds(0, rd)],
                        outd_h.at[pl.ds(c * NDP + s * rd, rd)])

    return k(ppd, pdd, znp)


def _make_gs(ch, src_stride, dst_stride, acc_rows):
    rp = acc_rows // 16
    nz = 8
    zr = rp // nz
    nbt = ch // IB

    def build(tbl, srci, dsti, zeros):
        @functools.partial(
            pl.kernel, mesh=_MESH,
            out_type=jax.ShapeDtypeStruct((2 * acc_rows, D2), jnp.float32),
            scratch_types=[pltpu.VMEM((2, IB, CK), jnp.int32),
                           pltpu.VMEM((2, IB, CK), jnp.int32),
                           pltpu.VMEM((2, CK, D2), jnp.float32),
                           pltpu.VMEM((zr, D2), jnp.float32),
                           pltpu.VMEM_SHARED((acc_rows, D2), jnp.float32),
                           pltpu.SemaphoreType.DMA,
                           pltpu.SemaphoreType.DMA,
                           pltpu.SemaphoreType.DMA,
                           pltpu.SemaphoreType.DMA,
                           pltpu.SemaphoreType.DMA],
            compiler_params=pltpu.CompilerParams(use_tc_tiling_on_sc=False),
        )
        def k(tbl_h, src_h, dst_h, z_h, out_h, srcb, dstb, rows, stg, acc,
              gs0, gs1, ss0, ss1, isem):
            c = lax.axis_index("c")
            s = lax.axis_index("s")
            pltpu.sync_copy(z_h, stg)
            for i in range(nz):
                pltpu.sync_copy(stg, acc.at[pl.ds(s * rp + i * zr, zr)])
            so = c * src_stride + s * ch
            do = c * dst_stride + s * ch
            pltpu.sync_copy(src_h.at[pl.ds(so, IB)], srcb.at[0])
            pltpu.sync_copy(dst_h.at[pl.ds(do, IB)], dstb.at[0])
            plsc.subcore_barrier()
            pltpu.async_copy(tbl_h.at[srcb.at[0].at[0]], rows.at[0], gs0)

            def batch2(bb2, carry):
                for bp in range(2):
                    bb = bb2 * 2 + bp
                    for kk in range(IB):
                        b = kk % 2
                        gsb = gs0 if b == 0 else gs1
                        gsn = gs1 if b == 0 else gs0
                        ssb = ss0 if b == 0 else ss1
                        ssn = ss1 if b == 0 else ss0
                        if kk == 0:
                            @pl.when(bb > 0)
                            def _():
                                pltpu.make_async_copy(
                                    rows.at[1 - b],
                                    acc.at[dstb.at[bp].at[kk]], ssn).wait()

                            @pl.when(bb + 1 < nbt)
                            def _():
                                pltpu.async_copy(
                                    src_h.at[pl.ds(so + (bb + 1) * IB, IB)],
                                    srcb.at[1 - bp], isem)
                                pltpu.async_copy(
                                    dst_h.at[pl.ds(do + (bb + 1) * IB, IB)],
                                    dstb.at[1 - bp], isem)
                        else:
                            pltpu.make_async_copy(
                                rows.at[1 - b],
                                acc.at[dstb.at[bp].at[kk]], ssn).wait()
                        if kk == IB - 1:
                            @pl.when(bb + 1 < nbt)
                            def _():
                                pltpu.make_async_copy(
                                    src_h.at[pl.ds(so + (bb + 1) * IB, IB)],
                                    srcb.at[1 - bp], isem).wait()
                                pltpu.make_async_copy(
                                    dst_h.at[pl.ds(do + (bb + 1) * IB, IB)],
                                    dstb.at[1 - bp], isem).wait()
                                pltpu.async_copy(
                                    tbl_h.at[srcb.at[1 - bp].at[0]],
                                    rows.at[1 - b], gsn)
                        else:
                            pltpu.async_copy(tbl_h.at[srcb.at[bp].at[kk + 1]],
                                             rows.at[1 - b], gsn)
                        pltpu.make_async_copy(tbl_h.at[srcb.at[bp].at[kk]],
                                              rows.at[b], gsb).wait()
                        pltpu.async_copy(rows.at[b],
                                         acc.at[dstb.at[bp].at[kk]], ssb,
                                         add=True)
                return carry

            lax.fori_loop(0, nbt // 2, batch2, 0)
            pltpu.make_async_copy(rows.at[1], acc.at[dstb.at[1].at[0]],
                                  ss1).wait()
            plsc.subcore_barrier()
            for i in range(nz):
                pltpu.sync_copy(acc.at[pl.ds(s * rp + i * zr, zr)], stg)
                pltpu.sync_copy(
                    stg, out_h.at[pl.ds(c * acc_rows + s * rp + i * zr, zr)])

        return k(tbl, srci, dsti, zeros)

    return build


_gs_l1 = _make_gs(PPCH // 16, PPCH, 0, NP)
_gs_l2 = _make_gs(PPCH // 32, PPCH // 2, PPCH // 2, NP)
_gs_pd = _make_gs(PDCH // 32, PDCH // 2, PDCH // 2, NDP)


def _tc1_body(x_ref, c0_ref, c1_ref, w_ref, g_ref, dinv_ref):
    drow = lax.rsqrt(1.0 + c0_ref[0, 0] + c1_ref[0, 0])
    dinv_ref[0, 0] = drow
    xs = x_ref[...] * drow.reshape(BM, 1)
    g_ref[0] = jnp.dot(xs, w_ref[0], preferred_element_type=jnp.float32)
    g_ref[1] = jnp.dot(xs, w_ref[1], preferred_element_type=jnp.float32)


def _tc1(x_pad, cntr, W1):
    nb = NP // BM
    f = pl.pallas_call(
        _tc1_body,
        grid=(nb,),
        in_specs=[
            pl.BlockSpec((BM, D_PROT), lambda j: (j, 0)),
            pl.BlockSpec((1, 1, BM), lambda j: (j, 0, 0)),
            pl.BlockSpec((1, 1, BM), lambda j: (nb + j, 0, 0)),
            pl.BlockSpec((2, D_PROT, D2), lambda j: (0, 0, 0)),
        ],
        out_specs=[
            pl.BlockSpec((2, BM, D2), lambda j: (0, j, 0)),
            pl.BlockSpec((1, 1, BM), lambda j: (j, 0, 0)),
        ],
        out_shape=[
            jax.ShapeDtypeStruct((2, NP, D2), jnp.float32),
            jax.ShapeDtypeStruct((nb, 1, BM), jnp.float32),
        ],
    )
    W1s = jnp.stack([W1[:, :D2], W1[:, D2:]])
    g3, dinvr = f(x_pad, cntr, cntr, W1s)
    return g3.reshape(2 * NP, D2), dinvr


def _tc2_body(s1a_ref, s1b_ref, g1a_ref, g1b_ref, d0_ref, d1_ref, b1_ref,
              w2_ref, g2_ref):
    dinv = jnp.concatenate([d0_ref[0, 0].reshape(BM, 1),
                            d1_ref[0, 0].reshape(BM, 1)], axis=0)
    ha = jax.nn.relu(dinv * (s1a_ref[...] + g1a_ref[...]) + b1_ref[0, 0])
    hb = jax.nn.relu(dinv * (s1b_ref[...] + g1b_ref[...]) + b1_ref[0, 1])
    g2_ref[...] = (jnp.dot(ha * dinv, w2_ref[0],
                           preferred_element_type=jnp.float32)
                   + jnp.dot(hb * dinv, w2_ref[1],
                             preferred_element_type=jnp.float32))


def _tc2(s1cat, g1cat, dinvr, b1, W2):
    bm2 = 2 * BM
    nb = NP // bm2
    f = pl.pallas_call(
        _tc2_body,
        grid=(nb,),
        in_specs=[
            pl.BlockSpec((bm2, D2), lambda j: (j, 0)),
            pl.BlockSpec((bm2, D2), lambda j: (nb + j, 0)),
            pl.BlockSpec((bm2, D2), lambda j: (j, 0)),
            pl.BlockSpec((bm2, D2), lambda j: (nb + j, 0)),
            pl.BlockSpec((1, 1, BM), lambda j: (2 * j, 0, 0)),
            pl.BlockSpec((1, 1, BM), lambda j: (2 * j + 1, 0, 0)),
            pl.BlockSpec((1, 2, D2), lambda j: (0, 0, 0)),
            pl.BlockSpec((2, D2, D2), lambda j: (0, 0, 0)),
        ],
        out_specs=pl.BlockSpec((bm2, D2), lambda j: (j, 0)),
        out_shape=jax.ShapeDtypeStruct((NP, D2), jnp.float32),
    )
    return f(s1cat, s1cat, g1cat, g1cat, dinvr, dinvr,
             b1.reshape(1, 2, D2), W2.reshape(2, D2, D2))


def _tc3_body(s2a_ref, s2b_ref, g2_ref, d0_ref, d1_ref, b2_ref, hf_ref):
    dinv = jnp.concatenate([d0_ref[0, 0].reshape(BM, 1),
                            d1_ref[0, 0].reshape(BM, 1)], axis=0)
    hf_ref[...] = dinv * (s2a_ref[...] + s2b_ref[...] + g2_ref[...]) \
        + b2_ref[...]


def _tc3(s2cat, g2, dinvr, b2):
    bm2 = 2 * BM
    nb = NP // bm2
    f = pl.pallas_call(
        _tc3_body,
        grid=(nb,),
        in_specs=[
            pl.BlockSpec((bm2, D2), lambda j: (j, 0)),
            pl.BlockSpec((bm2, D2), lambda j: (nb + j, 0)),
            pl.BlockSpec((bm2, D2), lambda j: (j, 0)),
            pl.BlockSpec((1, 1, BM), lambda j: (2 * j, 0, 0)),
            pl.BlockSpec((1, 1, BM), lambda j: (2 * j + 1, 0, 0)),
            pl.BlockSpec((1, D2), lambda j: (0, 0)),
        ],
        out_specs=pl.BlockSpec((bm2, D2), lambda j: (j, 0)),
        out_shape=jax.ShapeDtypeStruct((N_PROT, D2), jnp.float32),
    )
    return f(s2cat, s2cat, g2, dinvr, dinvr, b2.reshape(1, D2))


def _tc4_body(sa_ref, sb_ref, ca_ref, cb_ref, xd_ref, wl_ref, bl_ref, wr_ref,
              out_ref):
    crow = ca_ref[0, 0] + cb_ref[0, 0]
    cnt = jnp.maximum(crow, 1.0).reshape(BM, 1)
    mean = (sa_ref[...] + sb_ref[...]) / cnt
    out_ref[...] = (jnp.dot(mean, wl_ref[...],
                            preferred_element_type=jnp.float32)
                    + bl_ref[...]
                    + jnp.dot(xd_ref[...], wr_ref[...],
                              preferred_element_type=jnp.float32))


def _tc4(sumcat, cntdr, xd_pad, Wl, bl, Wr):
    nb = NDP // BM
    f = pl.pallas_call(
        _tc4_body,
        grid=(nb,),
        in_specs=[
            pl.BlockSpec((BM, D2), lambda j: (j, 0)),
            pl.BlockSpec((BM, D2), lambda j: (nb + j, 0)),
            pl.BlockSpec((1, 1, BM), lambda j: (j, 0, 0)),
            pl.BlockSpec((1, 1, BM), lambda j: (nb + j, 0, 0)),
            pl.BlockSpec((BM, D_PROT), lambda j: (j, 0)),
            pl.BlockSpec((D2, D_OUT), lambda j: (0, 0)),
            pl.BlockSpec((1, D_OUT), lambda j: (0, 0)),
            pl.BlockSpec((D_PROT, D_OUT), lambda j: (0, 0)),
        ],
        out_specs=pl.BlockSpec((BM, D_OUT), lambda j: (j, 0)),
        out_shape=jax.ShapeDtypeStruct((N_DRUG, D_OUT), jnp.float32),
    )
    return f(sumcat, sumcat, cntdr, cntdr, xd_pad, Wl, bl.reshape(1, D_OUT),
             Wr)


def kernel(x_prot, x_drug, pp_edge_index, pd_src, pd_dst,
           W1, b1, W2, b2, Wl, bl, Wr):
    src = pp_edge_index[0]
    dst = pp_edge_index[1]

    x_pad = jnp.zeros((NP, D_PROT), jnp.float32).at[:N_PROT].set(x_prot)
    xd_pad = jnp.zeros((NDP, D_PROT), jnp.float32).at[:N_DRUG].set(x_drug)

    padp = EP - E_PP
    trash_p = (N_PROT + (jnp.arange(padp, dtype=jnp.int32)
                         % (NP - N_PROT))).astype(jnp.int32)
    srcp = jnp.concatenate([src, trash_p]).reshape(PPCH, CK)
    dstp = jnp.concatenate([dst, trash_p]).reshape(PPCH, CK)
    src2 = jnp.concatenate([srcp, srcp + NP], axis=0)

    padd = EPD - E_PD
    trash_ds = (jnp.arange(padd, dtype=jnp.int32) % 176).astype(jnp.int32)
    trash_dd = (N_DRUG + (jnp.arange(padd, dtype=jnp.int32)
                          % (NDP - N_DRUG))).astype(jnp.int32)
    pdsrcp = jnp.concatenate([pd_src, trash_ds]).reshape(PDCH, CK)
    pddstp = jnp.concatenate([pd_dst, trash_dd]).reshape(PDCH, CK)

    znp1 = jnp.zeros((NP // 16,), jnp.float32)
    z32np = jnp.zeros((NP // 128, D2), jnp.float32)
    z32nd = jnp.zeros((NDP // 128, D2), jnp.float32)

    cntp, cntd = _sc_counts(dstp, pddstp, znp1)
    cntr = cntp.reshape(2 * NP // BM, 1, BM)

    g1cat, dinvr = _tc1(x_pad, cntr, W1)

    s1cat = _gs_l1(g1cat, src2, dstp, z32np)

    g2 = _tc2(s1cat, g1cat, dinvr, b1, W2)

    s2cat = _gs_l2(g2, srcp, dstp, z32np)

    hf = _tc3(s2cat, g2, dinvr, b2)

    sumcat = _gs_pd(hf, pdsrcp, pddstp, z32nd)

    cntdr = cntd.reshape(2 * NDP // BM, 1, BM)
    dti = _tc4(sumcat, cntdr, xd_pad, Wl, bl, Wr)

    return (hf, dti)
